# baseline (speedup 1.0000x reference)
import functools

import jax
import jax.numpy as jnp
from jax import lax
from jax.experimental import pallas as pl
from jax.experimental.pallas import tpu as pltpu
from jax.experimental.pallas import tpu_sc as plsc

D = 64
B = 1024
W = 128

NC = 2
NS = 16
NW = NC * NS
BPW = B // NW
L = 16

_mesh = plsc.VectorSubcoreMesh(core_axis_name="c", subcore_axis_name="s")


@functools.partial(
    pl.kernel,
    mesh=_mesh,
    out_type=[
        jax.ShapeDtypeStruct((B, W), jnp.float32),
        jax.ShapeDtypeStruct((B, W), jnp.float32),
        jax.ShapeDtypeStruct((B, W), jnp.float32),
    ],
    scratch_types=[
        pltpu.VMEM((B,), jnp.int32),
        pltpu.VMEM((B,), jnp.int32),
        pltpu.VMEM((BPW,), jnp.int32),
        pltpu.VMEM((BPW,), jnp.int32),
        pltpu.VMEM((BPW, W), jnp.float32),
        pltpu.VMEM((BPW, W), jnp.float32),
        pltpu.VMEM((BPW, W), jnp.float32),
        pltpu.SemaphoreType.DMA,
    ],
)
def _sc_gather(uid_hbm, iid_hbm, ta_hbm, tb_hbm,
               au_o, ai_o, bi_o,
               uids_v, iids_v, uidx_v, iidx_v, gau_v, gai_v, gb_v, sem):
    wid = lax.axis_index("s") * NC + lax.axis_index("c")
    base = pl.multiple_of(wid * BPW, BPW)
    pltpu.sync_copy(uid_hbm, uids_v)
    pltpu.sync_copy(iid_hbm, iids_v)
    for h in range(BPW // L):
        uidx_v[pl.ds(L * h, L)] = uids_v[pl.ds(base + L * h, L)]
        iidx_v[pl.ds(L * h, L)] = iids_v[pl.ds(base + L * h, L)]
    copies = [
        pltpu.async_copy(ta_hbm.at[uidx_v], gau_v, sem),
        pltpu.async_copy(ta_hbm.at[iidx_v], gai_v, sem),
        pltpu.async_copy(tb_hbm.at[iidx_v], gb_v, sem),
    ]
    for c in copies:
        c.wait()
    pltpu.sync_copy(gau_v, au_o.at[pl.ds(base, BPW)])
    pltpu.sync_copy(gai_v, ai_o.at[pl.ds(base, BPW)])
    pltpu.sync_copy(gb_v, bi_o.at[pl.ds(base, BPW)])


RB = 128


def _tc_body(au_ref, ai_ref, gb_ref, o_ref):
    au = au_ref[...]
    gb = gb_ref[...]
    ue = au[:, :D]
    ie = gb[:, :D]
    cie = gb[:, D:]
    ones = jnp.ones((1, D), dtype=jnp.float32)
    cd = lax.dot_general(ones, ue * cie, (((1,), (1,)), ((), ())),
                         precision=lax.Precision.HIGHEST,
                         preferred_element_type=jnp.float32)
    rd = lax.dot_general(ones, ue * ie, (((1,), (1,)), ((), ())),
                         precision=lax.Precision.HIGHEST,
                         preferred_element_type=jnp.float32)
    i = pl.program_id(0)
    aub = au_ref[pl.ds(i * RB, RB), :]
    aib = ai_ref[pl.ds(i * RB, RB), :]
    bc = aub[:, D + 1:D + 2] + aib[:, D + 3:D + 4]
    br = aub[:, D:D + 1] + aib[:, D + 2:D + 3]
    obs = 1.0 / (1.0 + jnp.exp(-(cd + bc)))
    o_ref[...] = obs * (rd + br)


_tc_map = pl.pallas_call(
    _tc_body,
    grid=(B // RB,),
    in_specs=[
        pl.BlockSpec((B, W), lambda i: (0, 0)),
        pl.BlockSpec((B, W), lambda i: (0, 0)),
        pl.BlockSpec((B, W), lambda i: (0, 0)),
    ],
    out_specs=pl.BlockSpec((RB, B), lambda i: (i, 0)),
    out_shape=jax.ShapeDtypeStruct((B, B), jnp.float32),
)


def kernel(user_ids, item_ids, user_emb, item_emb, cens_item_emb,
           user_bias, item_bias, cens_user_bias, cens_item_bias):
    uid = user_ids.astype(jnp.int32)
    iid = item_ids.astype(jnp.int32)
    ta = jnp.concatenate(
        [user_emb, user_bias, cens_user_bias, item_bias, cens_item_bias],
        axis=1)
    ta = jnp.pad(ta, ((0, 0), (0, W - D - 4)))
    tb = jnp.concatenate([item_emb, cens_item_emb], axis=1)
    au, ai, gb = _sc_gather(uid, iid, ta, tb)
    return _tc_map(au, ai, gb)


# speedup vs baseline: 2.5513x; 2.5513x over previous
import functools

import jax
import jax.numpy as jnp
from jax import lax
from jax.experimental import pallas as pl
from jax.experimental.pallas import tpu as pltpu
from jax.experimental.pallas import tpu_sc as plsc

N = 100000
D = 64
B = 1024
W = 128

NC = 2
NS = 16
NW = NC * NS
BPW = B // NW
L = 16
G = 2048
GC = G // W

_mesh = plsc.VectorSubcoreMesh(core_axis_name="c", subcore_axis_name="s")


@functools.partial(
    pl.kernel,
    mesh=_mesh,
    out_type=[jax.ShapeDtypeStruct((2 * B * W,), jnp.float32),
              jax.ShapeDtypeStruct((B,), jnp.float32),
              jax.ShapeDtypeStruct((B,), jnp.float32)],
    scratch_types=[
        pltpu.VMEM((BPW,), jnp.int32),
        pltpu.VMEM((BPW,), jnp.int32),
        pltpu.VMEM((G,), jnp.int32),
        pltpu.VMEM((G,), jnp.int32),
        pltpu.VMEM((G,), jnp.float32),
        pltpu.VMEM((G,), jnp.float32),
        pltpu.VMEM((G,), jnp.float32),
        pltpu.VMEM((BPW,), jnp.float32),
        pltpu.VMEM((BPW,), jnp.float32),
        pltpu.VMEM((BPW,), jnp.float32),
        pltpu.VMEM((BPW,), jnp.float32),
        pltpu.VMEM((2 * BPW * W,), jnp.float32),
        pltpu.VMEM((BPW,), jnp.float32),
        pltpu.VMEM((BPW,), jnp.float32),
        pltpu.SemaphoreType.DMA,
    ],
)
def _sc_gather(uid_hbm, iid_hbm, uef_hbm, ief_hbm, cief_hbm,
               ubf_hbm, ibf_hbm, cubf_hbm, cibf_hbm,
               out_hbm, bc_o, br_o,
               uidx_v, iidx_v, fidxu_v, fidxi_v, gue_v, gie_v, gcie_v,
               ub_v, ib_v, cub_v, cib_v, pk_v, bc_v, br_v, sem):
    wid = lax.axis_index("s") * NC + lax.axis_index("c")
    base = pl.multiple_of(wid * BPW, BPW)
    pltpu.sync_copy(uid_hbm.at[pl.ds(base, BPW)], uidx_v)
    pltpu.sync_copy(iid_hbm.at[pl.ds(base, BPW)], iidx_v)
    dstep = lax.mul(lax.iota(jnp.int32, L), N)
    for h in range(BPW // L):
        uv = uidx_v[pl.ds(L * h, L)]
        iv = iidx_v[pl.ds(L * h, L)]
        for j in range(L):
            k = L * h + j
            for q in range(D // L):
                sl = pl.ds(D * k + L * q, L)
                fidxu_v[sl] = dstep + (uv[j] + q * (L * N))
                fidxi_v[sl] = dstep + (iv[j] + q * (L * N))
    copies = []
    for m in range(GC):
        sl = pl.ds(W * m, W)
        copies.append(pltpu.async_copy(uef_hbm.at[fidxu_v.at[sl]],
                                       gue_v.at[sl], sem))
        copies.append(pltpu.async_copy(ief_hbm.at[fidxi_v.at[sl]],
                                       gie_v.at[sl], sem))
        copies.append(pltpu.async_copy(cief_hbm.at[fidxi_v.at[sl]],
                                       gcie_v.at[sl], sem))
    copies.append(pltpu.async_copy(ubf_hbm.at[uidx_v], ub_v, sem))
    copies.append(pltpu.async_copy(ibf_hbm.at[iidx_v], ib_v, sem))
    copies.append(pltpu.async_copy(cubf_hbm.at[uidx_v], cub_v, sem))
    copies.append(pltpu.async_copy(cibf_hbm.at[iidx_v], cib_v, sem))
    for c in copies:
        c.wait()
    for k in range(BPW):
        for q in range(D // L):
            src = pl.ds(D * k + L * q, L)
            pk_v[pl.ds(W * k + L * q, L)] = gue_v[src]
            pk_v[pl.ds(W * k + D + L * q, L)] = gie_v[src]
            pk_v[pl.ds(W * (BPW + k) + L * q, L)] = gcie_v[src]
    for h in range(BPW // L):
        sl = pl.ds(L * h, L)
        bc_v[sl] = cub_v[sl] + cib_v[sl]
        br_v[sl] = ub_v[sl] + ib_v[sl]
    pltpu.sync_copy(bc_v, bc_o.at[pl.ds(base, BPW)])
    pltpu.sync_copy(br_v, br_o.at[pl.ds(base, BPW)])
    pltpu.sync_copy(pk_v.at[pl.ds(0, BPW * W)],
                    out_hbm.at[pl.ds(base * W, BPW * W)])
    pltpu.sync_copy(pk_v.at[pl.ds(BPW * W, BPW * W)],
                    out_hbm.at[pl.ds((B + base) * W, BPW * W)])


RB = 128


def _tc_body(x1_ref, x2_ref, bc_ref, br_ref, o_ref):
    x1 = x1_ref[...]
    x2 = x2_ref[...]
    ue = x1[:, :D]
    ie = x1[:, D:]
    cie = x2[:, :D]
    ones = jnp.ones((1, D), dtype=jnp.float32)
    cd = lax.dot_general(ones, ue * cie, (((1,), (1,)), ((), ())),
                         precision=lax.Precision.HIGHEST,
                         preferred_element_type=jnp.float32)
    rd = lax.dot_general(ones, ue * ie, (((1,), (1,)), ((), ())),
                         precision=lax.Precision.HIGHEST,
                         preferred_element_type=jnp.float32)
    bc = bc_ref[...]
    br = br_ref[...]
    obs = 1.0 / (1.0 + jnp.exp(-(cd + bc)))
    o_ref[...] = obs * (rd + br)


_tc_map = pl.pallas_call(
    _tc_body,
    grid=(B // RB,),
    in_specs=[
        pl.BlockSpec((B, W), lambda i: (0, 0)),
        pl.BlockSpec((B, W), lambda i: (1, 0)),
        pl.BlockSpec((RB, 1), lambda i: (i, 0)),
        pl.BlockSpec((RB, 1), lambda i: (i, 0)),
    ],
    out_specs=pl.BlockSpec((RB, B), lambda i: (i, 0)),
    out_shape=jax.ShapeDtypeStruct((B, B), jnp.float32),
)


def kernel(user_ids, item_ids, user_emb, item_emb, cens_item_emb,
           user_bias, item_bias, cens_user_bias, cens_item_bias):
    uid = user_ids.astype(jnp.int32)
    iid = item_ids.astype(jnp.int32)
    packed, bc, br = _sc_gather(
        uid, iid,
        user_emb.T.reshape(-1), item_emb.T.reshape(-1),
        cens_item_emb.T.reshape(-1),
        user_bias.reshape(-1), item_bias.reshape(-1),
        cens_user_bias.reshape(-1), cens_item_bias.reshape(-1))
    x = packed.reshape(2 * B, W)
    return _tc_map(x, x, bc.reshape(B, 1), br.reshape(B, 1))


# speedup vs baseline: 3.4443x; 1.3500x over previous
import functools

import jax
import jax.numpy as jnp
from jax import lax
from jax.experimental import pallas as pl
from jax.experimental.pallas import tpu as pltpu
from jax.experimental.pallas import tpu_sc as plsc

N = 100000
D = 64
B = 1024
W = 128

NC = 2
NS = 16
NW = NC * NS
BPW = B // NW
L = 16
G = 2048
GC = G // W

_mesh = plsc.VectorSubcoreMesh(core_axis_name="c", subcore_axis_name="s")

NCOL = 782
SLAB = 8 * W * 8
FLAT = NCOL * SLAB
CPW = 25
CHUNK = 8


@functools.partial(
    pl.kernel,
    mesh=_mesh,
    out_type=[jax.ShapeDtypeStruct((NCOL, D, W), jnp.float32)] * 3,
    scratch_types=[
        pltpu.VMEM((CHUNK, D, W), jnp.float32),
        pltpu.SemaphoreType.DMA,
    ],
)
def _sc_detile(ueT_hbm, ieT_hbm, cieT_hbm, uef_o, ief_o, cief_o, slab_v, sem):
    wid = lax.axis_index("s") * NC + lax.axis_index("c")
    start = wid * CPW
    for src, dst in ((ueT_hbm, uef_o), (ieT_hbm, ief_o), (cieT_hbm, cief_o)):
        for c0 in range(0, CPW, CHUNK):
            nb = min(CHUNK, CPW - c0)
            cs = [jnp.minimum(start + c0 + b, NCOL - 1) for b in range(nb)]
            reads = [
                pltpu.async_copy(
                    src.at[:, pl.ds(pl.multiple_of(cs[b] * W, W), W)],
                    slab_v.at[b], sem)
                for b in range(nb)
            ]
            for r in reads:
                r.wait()
            for b in range(nb):
                pltpu.sync_copy(slab_v.at[b], dst.at[cs[b]])


@functools.partial(
    pl.kernel,
    mesh=_mesh,
    out_type=[jax.ShapeDtypeStruct((2 * B * W,), jnp.float32),
              jax.ShapeDtypeStruct((B,), jnp.float32),
              jax.ShapeDtypeStruct((B,), jnp.float32)],
    scratch_types=[
        pltpu.VMEM((BPW,), jnp.int32),
        pltpu.VMEM((BPW,), jnp.int32),
        pltpu.VMEM((G,), jnp.int32),
        pltpu.VMEM((G,), jnp.int32),
        pltpu.VMEM((G,), jnp.float32),
        pltpu.VMEM((G,), jnp.float32),
        pltpu.VMEM((G,), jnp.float32),
        pltpu.VMEM((BPW,), jnp.float32),
        pltpu.VMEM((BPW,), jnp.float32),
        pltpu.VMEM((BPW,), jnp.float32),
        pltpu.VMEM((BPW,), jnp.float32),
        pltpu.VMEM((2 * BPW * W,), jnp.float32),
        pltpu.VMEM((BPW,), jnp.float32),
        pltpu.VMEM((BPW,), jnp.float32),
        pltpu.SemaphoreType.DMA,
    ],
)
def _sc_gather(uid_hbm, iid_hbm, uef_hbm, ief_hbm, cief_hbm,
               ubf_hbm, ibf_hbm, cubf_hbm, cibf_hbm,
               out_hbm, bc_o, br_o,
               uidx_v, iidx_v, fidxu_v, fidxi_v, gue_v, gie_v, gcie_v,
               ub_v, ib_v, cub_v, cib_v, pk_v, bc_v, br_v, sem):
    wid = lax.axis_index("s") * NC + lax.axis_index("c")
    base = pl.multiple_of(wid * BPW, BPW)
    pltpu.sync_copy(uid_hbm.at[pl.ds(base, BPW)], uidx_v)
    pltpu.sync_copy(iid_hbm.at[pl.ds(base, BPW)], iidx_v)
    dstep = lax.mul(lax.iota(jnp.int32, L), W)
    for h in range(BPW // L):
        uv = uidx_v[pl.ds(L * h, L)]
        iv = iidx_v[pl.ds(L * h, L)]
        ub_ = (lax.shift_right_logical(uv, 7) * SLAB
               + lax.bitwise_and(uv, W - 1))
        ib_ = (lax.shift_right_logical(iv, 7) * SLAB
               + lax.bitwise_and(iv, W - 1))
        for j in range(L):
            k = L * h + j
            for q in range(D // L):
                sl = pl.ds(D * k + L * q, L)
                fidxu_v[sl] = dstep + (ub_[j] + q * (L * W))
                fidxi_v[sl] = dstep + (ib_[j] + q * (L * W))
    copies = []
    for m in range(GC):
        sl = pl.ds(W * m, W)
        copies.append(pltpu.async_copy(uef_hbm.at[fidxu_v.at[sl]],
                                       gue_v.at[sl], sem))
        copies.append(pltpu.async_copy(ief_hbm.at[fidxi_v.at[sl]],
                                       gie_v.at[sl], sem))
        copies.append(pltpu.async_copy(cief_hbm.at[fidxi_v.at[sl]],
                                       gcie_v.at[sl], sem))
    copies.append(pltpu.async_copy(ubf_hbm.at[uidx_v], ub_v, sem))
    copies.append(pltpu.async_copy(ibf_hbm.at[iidx_v], ib_v, sem))
    copies.append(pltpu.async_copy(cubf_hbm.at[uidx_v], cub_v, sem))
    copies.append(pltpu.async_copy(cibf_hbm.at[iidx_v], cib_v, sem))
    for c in copies:
        c.wait()
    for k in range(BPW):
        for q in range(D // L):
            src = pl.ds(D * k + L * q, L)
            pk_v[pl.ds(W * k + L * q, L)] = gue_v[src]
            pk_v[pl.ds(W * k + D + L * q, L)] = gie_v[src]
            pk_v[pl.ds(W * (BPW + k) + L * q, L)] = gcie_v[src]
    for h in range(BPW // L):
        sl = pl.ds(L * h, L)
        bc_v[sl] = cub_v[sl] + cib_v[sl]
        br_v[sl] = ub_v[sl] + ib_v[sl]
    pltpu.sync_copy(bc_v, bc_o.at[pl.ds(base, BPW)])
    pltpu.sync_copy(br_v, br_o.at[pl.ds(base, BPW)])
    pltpu.sync_copy(pk_v.at[pl.ds(0, BPW * W)],
                    out_hbm.at[pl.ds(base * W, BPW * W)])
    pltpu.sync_copy(pk_v.at[pl.ds(BPW * W, BPW * W)],
                    out_hbm.at[pl.ds((B + base) * W, BPW * W)])


RB = 128


def _tc_body(x1_ref, x2_ref, bc_ref, br_ref, o_ref):
    x1 = x1_ref[...]
    x2 = x2_ref[...]
    ue = x1[:, :D]
    ie = x1[:, D:]
    cie = x2[:, :D]
    ones = jnp.ones((1, D), dtype=jnp.float32)
    cd = lax.dot_general(ones, ue * cie, (((1,), (1,)), ((), ())),
                         precision=lax.Precision.HIGHEST,
                         preferred_element_type=jnp.float32)
    rd = lax.dot_general(ones, ue * ie, (((1,), (1,)), ((), ())),
                         precision=lax.Precision.HIGHEST,
                         preferred_element_type=jnp.float32)
    bc = bc_ref[...]
    br = br_ref[...]
    obs = 1.0 / (1.0 + jnp.exp(-(cd + bc)))
    o_ref[...] = obs * (rd + br)


_tc_map = pl.pallas_call(
    _tc_body,
    grid=(B // RB,),
    in_specs=[
        pl.BlockSpec((B, W), lambda i: (0, 0)),
        pl.BlockSpec((B, W), lambda i: (1, 0)),
        pl.BlockSpec((RB, 1), lambda i: (i, 0)),
        pl.BlockSpec((RB, 1), lambda i: (i, 0)),
    ],
    out_specs=pl.BlockSpec((RB, B), lambda i: (i, 0)),
    out_shape=jax.ShapeDtypeStruct((B, B), jnp.float32),
)


def kernel(user_ids, item_ids, user_emb, item_emb, cens_item_emb,
           user_bias, item_bias, cens_user_bias, cens_item_bias):
    uid = user_ids.astype(jnp.int32)
    iid = item_ids.astype(jnp.int32)
    uef, ief, cief = _sc_detile(user_emb.T, item_emb.T, cens_item_emb.T)
    packed, bc, br = _sc_gather(
        uid, iid,
        uef.reshape(-1), ief.reshape(-1), cief.reshape(-1),
        user_bias.reshape(-1), item_bias.reshape(-1),
        cens_user_bias.reshape(-1), cens_item_bias.reshape(-1))
    x = packed.reshape(2 * B, W)
    return _tc_map(x, x, bc.reshape(B, 1), br.reshape(B, 1))


# speedup vs baseline: 3.6632x; 1.0635x over previous
import functools

import jax
import jax.numpy as jnp
from jax import lax
from jax.experimental import pallas as pl
from jax.experimental.pallas import tpu as pltpu
from jax.experimental.pallas import tpu_sc as plsc

N = 100000
D = 64
B = 1024
W = 128

NC = 2
NS = 16
NW = NC * NS
BPW = B // NW
L = 16
G = 2048
GC = G // W

_mesh = plsc.VectorSubcoreMesh(core_axis_name="c", subcore_axis_name="s")

NCOL = 782
SLAB = D * W
FLAT = NCOL * SLAB
CPW = 25
CHUNK = 7


@functools.partial(
    pl.kernel,
    mesh=_mesh,
    out_type=[jax.ShapeDtypeStruct((NCOL, D, W), jnp.float32)] * 3,
    scratch_types=[
        pltpu.VMEM((2, CHUNK, D, W), jnp.float32),
        pltpu.SemaphoreType.DMA,
        pltpu.SemaphoreType.DMA,
    ],
)
def _sc_detile(ueT_hbm, ieT_hbm, cieT_hbm, uef_o, ief_o, cief_o,
               slab_v, semr, semw):
    wid = lax.axis_index("s") * NC + lax.axis_index("c")
    start = wid * CPW
    tables = ((ueT_hbm, uef_o), (ieT_hbm, ief_o), (cieT_hbm, cief_o))
    jobs = [(t, c0, min(CHUNK, CPW - c0))
            for t in range(3) for c0 in range(0, CPW, CHUNK)]

    def fire_reads(i, buf):
        src = tables[jobs[i][0]][0]
        c0, nb = jobs[i][1], jobs[i][2]
        cs = [jnp.minimum(start + c0 + b, NCOL - 1) for b in range(nb)]
        return cs, [
            pltpu.async_copy(
                src.at[:, pl.ds(pl.multiple_of(cs[b] * W, W), W)],
                slab_v.at[buf, b], semr)
            for b in range(nb)
        ]

    def fire_writes(i, buf, cs):
        dst = tables[jobs[i][0]][1]
        nb = jobs[i][2]
        return [
            pltpu.async_copy(slab_v.at[buf, b], dst.at[cs[b]], semw)
            for b in range(nb)
        ]

    cs_cur, reads_cur = fire_reads(0, 0)
    writes_prev = None
    for i in range(len(jobs)):
        buf = i % 2
        if writes_prev is not None:
            for wd in writes_prev:
                wd.wait()
        if i + 1 < len(jobs):
            cs_nxt, reads_nxt = fire_reads(i + 1, 1 - buf)
        for rd in reads_cur:
            rd.wait()
        writes_cur = fire_writes(i, buf, cs_cur)
        writes_prev = writes_cur
        if i + 1 < len(jobs):
            cs_cur, reads_cur = cs_nxt, reads_nxt
    for wd in writes_prev:
        wd.wait()


@functools.partial(
    pl.kernel,
    mesh=_mesh,
    out_type=[jax.ShapeDtypeStruct((2 * B * W,), jnp.float32),
              jax.ShapeDtypeStruct((B,), jnp.float32),
              jax.ShapeDtypeStruct((B,), jnp.float32)],
    scratch_types=[
        pltpu.VMEM((BPW,), jnp.int32),
        pltpu.VMEM((BPW,), jnp.int32),
        pltpu.VMEM((G,), jnp.int32),
        pltpu.VMEM((G,), jnp.int32),
        pltpu.VMEM((G,), jnp.float32),
        pltpu.VMEM((G,), jnp.float32),
        pltpu.VMEM((G,), jnp.float32),
        pltpu.VMEM((BPW,), jnp.float32),
        pltpu.VMEM((BPW,), jnp.float32),
        pltpu.VMEM((BPW,), jnp.float32),
        pltpu.VMEM((BPW,), jnp.float32),
        pltpu.VMEM((2 * BPW * W,), jnp.float32),
        pltpu.VMEM((BPW,), jnp.float32),
        pltpu.VMEM((BPW,), jnp.float32),
        pltpu.SemaphoreType.DMA,
    ],
)
def _sc_gather(uid_hbm, iid_hbm, uef_hbm, ief_hbm, cief_hbm,
               ubf_hbm, ibf_hbm, cubf_hbm, cibf_hbm,
               out_hbm, bc_o, br_o,
               uidx_v, iidx_v, fidxu_v, fidxi_v, gue_v, gie_v, gcie_v,
               ub_v, ib_v, cub_v, cib_v, pk_v, bc_v, br_v, sem):
    wid = lax.axis_index("s") * NC + lax.axis_index("c")
    base = pl.multiple_of(wid * BPW, BPW)
    pltpu.sync_copy(uid_hbm.at[pl.ds(base, BPW)], uidx_v)
    pltpu.sync_copy(iid_hbm.at[pl.ds(base, BPW)], iidx_v)
    dstep = lax.mul(lax.iota(jnp.int32, L), W)
    for h in range(BPW // L):
        uv = uidx_v[pl.ds(L * h, L)]
        iv = iidx_v[pl.ds(L * h, L)]
        ub_ = (lax.shift_right_logical(uv, 7) * SLAB
               + lax.bitwise_and(uv, W - 1))
        ib_ = (lax.shift_right_logical(iv, 7) * SLAB
               + lax.bitwise_and(iv, W - 1))
        for j in range(L):
            k = L * h + j
            for q in range(D // L):
                sl = pl.ds(D * k + L * q, L)
                fidxu_v[sl] = dstep + (ub_[j] + q * (L * W))
                fidxi_v[sl] = dstep + (ib_[j] + q * (L * W))
    copies = []
    for m in range(GC):
        sl = pl.ds(W * m, W)
        copies.append(pltpu.async_copy(uef_hbm.at[fidxu_v.at[sl]],
                                       gue_v.at[sl], sem))
        copies.append(pltpu.async_copy(ief_hbm.at[fidxi_v.at[sl]],
                                       gie_v.at[sl], sem))
        copies.append(pltpu.async_copy(cief_hbm.at[fidxi_v.at[sl]],
                                       gcie_v.at[sl], sem))
    copies.append(pltpu.async_copy(ubf_hbm.at[uidx_v], ub_v, sem))
    copies.append(pltpu.async_copy(ibf_hbm.at[iidx_v], ib_v, sem))
    copies.append(pltpu.async_copy(cubf_hbm.at[uidx_v], cub_v, sem))
    copies.append(pltpu.async_copy(cibf_hbm.at[iidx_v], cib_v, sem))
    for c in copies:
        c.wait()
    for k in range(BPW):
        for q in range(D // L):
            src = pl.ds(D * k + L * q, L)
            pk_v[pl.ds(W * k + L * q, L)] = gue_v[src]
            pk_v[pl.ds(W * k + D + L * q, L)] = gie_v[src]
            pk_v[pl.ds(W * (BPW + k) + L * q, L)] = gcie_v[src]
    for h in range(BPW // L):
        sl = pl.ds(L * h, L)
        bc_v[sl] = cub_v[sl] + cib_v[sl]
        br_v[sl] = ub_v[sl] + ib_v[sl]
    pltpu.sync_copy(bc_v, bc_o.at[pl.ds(base, BPW)])
    pltpu.sync_copy(br_v, br_o.at[pl.ds(base, BPW)])
    pltpu.sync_copy(pk_v.at[pl.ds(0, BPW * W)],
                    out_hbm.at[pl.ds(base * W, BPW * W)])
    pltpu.sync_copy(pk_v.at[pl.ds(BPW * W, BPW * W)],
                    out_hbm.at[pl.ds((B + base) * W, BPW * W)])


RB = 128


def _tc_body(x1_ref, x2_ref, bc_ref, br_ref, o_ref):
    x1 = x1_ref[...]
    x2 = x2_ref[...]
    ue = x1[:, :D]
    ie = x1[:, D:]
    cie = x2[:, :D]
    ones = jnp.ones((1, D), dtype=jnp.float32)
    cd = lax.dot_general(ones, ue * cie, (((1,), (1,)), ((), ())),
                         precision=lax.Precision.HIGHEST,
                         preferred_element_type=jnp.float32)
    rd = lax.dot_general(ones, ue * ie, (((1,), (1,)), ((), ())),
                         precision=lax.Precision.HIGHEST,
                         preferred_element_type=jnp.float32)
    bc = bc_ref[...]
    br = br_ref[...]
    obs = 1.0 / (1.0 + jnp.exp(-(cd + bc)))
    o_ref[...] = obs * (rd + br)


_tc_map = pl.pallas_call(
    _tc_body,
    grid=(B // RB,),
    in_specs=[
        pl.BlockSpec((B, W), lambda i: (0, 0)),
        pl.BlockSpec((B, W), lambda i: (1, 0)),
        pl.BlockSpec((RB, 1), lambda i: (i, 0)),
        pl.BlockSpec((RB, 1), lambda i: (i, 0)),
    ],
    out_specs=pl.BlockSpec((RB, B), lambda i: (i, 0)),
    out_shape=jax.ShapeDtypeStruct((B, B), jnp.float32),
)


def kernel(user_ids, item_ids, user_emb, item_emb, cens_item_emb,
           user_bias, item_bias, cens_user_bias, cens_item_bias):
    uid = user_ids.astype(jnp.int32)
    iid = item_ids.astype(jnp.int32)
    uef, ief, cief = _sc_detile(user_emb.T, item_emb.T, cens_item_emb.T)
    packed, bc, br = _sc_gather(
        uid, iid,
        uef.reshape(-1), ief.reshape(-1), cief.reshape(-1),
        user_bias.reshape(-1), item_bias.reshape(-1),
        cens_user_bias.reshape(-1), cens_item_bias.reshape(-1))
    x = packed.reshape(2 * B, W)
    return _tc_map(x, x, bc.reshape(B, 1), br.reshape(B, 1))
